# Initial kernel scaffold; baseline (speedup 1.0000x reference)
import jax
import jax.numpy as jnp
from jax.experimental import pallas as pl


def kernel(x_obs, actions, eps, W_enc1, b_enc1, W_mean, b_mean, W_std, b_std, W_st, W_sig1, b_sig1, W_sig2, b_sig2, W_dec1, b_dec1, W_dec2, b_dec2):
    raise NotImplementedError("write your pallas kernel here")


# speedup vs baseline: 21.3331x; 21.3331x over previous
import functools

import jax
import jax.numpy as jnp
from jax.experimental import pallas as pl
from jax.experimental.pallas import tpu as pltpu

B = 64
OBS = 2048
TOTAL = 2304
PRED = TOTAL - OBS
A_DIM = 5
S_DIM = 2
Z_DIM = 16
K = 5
DELTA = 1e-4
H = 512
XFLAT = 192

ENC_TILE = 1024


def _enc_kernel(x_ref, w1_ref, b1_ref, wms_ref, bms_ref, out_ref):
    x = x_ref[...] * 2.0 - 1.0
    h = jnp.tanh(jnp.dot(x, w1_ref[...], preferred_element_type=jnp.float32)
                 + b1_ref[...])
    y = jnp.dot(h, wms_ref[...], preferred_element_type=jnp.float32) + bms_ref[...]
    col = jax.lax.broadcasted_iota(jnp.int32, y.shape, 1)
    out_ref[...] = jnp.where(col < Z_DIM, y, jnp.exp(y))


def _scan_kernel(act_ref, wst_ref, w1_ref, b1_ref, w2_ref, b2_ref,
                 stx_ref, sty_ref, rx_ref, ry_ref):
    rx = jnp.zeros((TOTAL, B), jnp.float32)
    ry = jnp.zeros((TOTAL, B), jnp.float32)
    for a in range(A_DIM):
        act_a = act_ref[a]
        rx = rx + act_a * wst_ref[a, 0]
        ry = ry + act_a * wst_ref[a, 1]
    rx_ref[...] = rx
    ry_ref[...] = ry
    stx_ref[0:1, :] = jnp.zeros((1, B), jnp.float32)
    sty_ref[0:1, :] = jnp.zeros((1, B), jnp.float32)

    def step(t, st):
        stx, sty = st
        rpx = rx_ref[pl.ds(t, 1), :]
        rpy = ry_ref[pl.ds(t, 1), :]
        spx = stx + rpx
        spy = sty + rpy
        gx = jnp.zeros((1, B), jnp.float32) + b2_ref[0, 0]
        gy = jnp.zeros((1, B), jnp.float32) + b2_ref[0, 1]
        for k in range(10):
            hk = jnp.tanh(spx * w1_ref[0, k] + spy * w1_ref[1, k]
                          + b1_ref[0, k])
            gx = gx + hk * w2_ref[k, 0]
            gy = gy + hk * w2_ref[k, 1]
        stx = stx + rpx * jax.nn.sigmoid(gx)
        sty = sty + rpy * jax.nn.sigmoid(gy)
        stx_ref[pl.ds(t, 1), :] = stx
        sty_ref[pl.ds(t, 1), :] = sty
        return stx, sty

    jax.lax.fori_loop(1, TOTAL, step,
                      (jnp.zeros((1, B), jnp.float32),
                       jnp.zeros((1, B), jnp.float32)))


def _knn_dec_kernel(q_ref, m_ref, zt_ref, eps_ref, wd1_ref, bd1_ref,
                    wd2_ref, bd2_ref, out_ref):
    q = q_ref[0]
    mt = m_ref[0]
    qx = q[:, 0:1]
    qy = q[:, 1:2]
    mx = mt[0:1, :]
    my = mt[1:2, :]
    dx = qx - mx
    dy = qy - my
    d = dx * dx + dy * dy

    col = jax.lax.broadcasted_iota(jnp.int32, (PRED, OBS), 1)
    u = jnp.zeros((PRED, OBS), jnp.float32)
    denom = jnp.zeros((PRED, 1), jnp.float32)
    for _ in range(K):
        mn = jnp.min(d, axis=1, keepdims=True)
        idx = jnp.min(jnp.where(d <= mn, col, OBS), axis=1, keepdims=True)
        oneh = col == idx
        wk = 1.0 / (mn + DELTA)
        u = u + jnp.where(oneh, wk, 0.0)
        denom = denom + wk
        d = jnp.where(oneh, jnp.float32(1e30), d)

    zt = zt_ref[0]
    numer = jnp.dot(u, zt, preferred_element_type=jnp.float32)
    mean = numer[:, :Z_DIM] / denom
    std = numer[:, Z_DIM:] / denom
    z = mean + std * eps_ref[0]
    hd = jnp.tanh(jnp.dot(z, wd1_ref[...], preferred_element_type=jnp.float32)
                  + bd1_ref[...])
    xr = jnp.tanh(jnp.dot(hd, wd2_ref[...], preferred_element_type=jnp.float32)
                  + bd2_ref[...])
    out_ref[0] = (xr + 1.0) * 0.5


@jax.jit
def kernel(x_obs, actions, eps, W_enc1, b_enc1, W_mean, b_mean, W_std, b_std,
           W_st, W_sig1, b_sig1, W_sig2, b_sig2, W_dec1, b_dec1, W_dec2, b_dec2):
    f32 = jnp.float32
    nrows = B * OBS
    x_flat = x_obs.reshape(nrows, XFLAT)
    w_ms = jnp.concatenate([W_mean, W_std], axis=1)
    b_ms = jnp.concatenate([b_mean, b_std], axis=0).reshape(1, 2 * Z_DIM)
    b1r = b_enc1.reshape(1, H)

    zt = pl.pallas_call(
        _enc_kernel,
        grid=(nrows // ENC_TILE,),
        in_specs=[
            pl.BlockSpec((ENC_TILE, XFLAT), lambda i: (i, 0)),
            pl.BlockSpec((XFLAT, H), lambda i: (0, 0)),
            pl.BlockSpec((1, H), lambda i: (0, 0)),
            pl.BlockSpec((H, 2 * Z_DIM), lambda i: (0, 0)),
            pl.BlockSpec((1, 2 * Z_DIM), lambda i: (0, 0)),
        ],
        out_specs=pl.BlockSpec((ENC_TILE, 2 * Z_DIM), lambda i: (i, 0)),
        out_shape=jax.ShapeDtypeStruct((nrows, 2 * Z_DIM), f32),
    )(x_flat, W_enc1, b1r, w_ms, b_ms)
    zt = zt.reshape(B, OBS, 2 * Z_DIM)

    act_r = actions.transpose(1, 2, 0)
    smem_spec = pl.BlockSpec(memory_space=pltpu.SMEM)
    st_x, st_y = pl.pallas_call(
        _scan_kernel,
        grid=(1,),
        in_specs=[
            pl.BlockSpec((A_DIM, TOTAL, B), lambda i: (0, 0, 0)),
            smem_spec, smem_spec, smem_spec, smem_spec, smem_spec,
        ],
        out_specs=[pl.BlockSpec((TOTAL, B), lambda i: (0, 0)),
                   pl.BlockSpec((TOTAL, B), lambda i: (0, 0))],
        out_shape=[jax.ShapeDtypeStruct((TOTAL, B), f32),
                   jax.ShapeDtypeStruct((TOTAL, B), f32)],
        scratch_shapes=[pltpu.VMEM((TOTAL, B), f32),
                        pltpu.VMEM((TOTAL, B), f32)],
    )(act_r, W_st, W_sig1, b_sig1.reshape(1, 10), W_sig2,
      b_sig2.reshape(1, S_DIM))

    q = jnp.stack([st_x[OBS:], st_y[OBS:]], axis=-1).transpose(1, 0, 2)
    m_t = jnp.stack([st_x[:OBS].T, st_y[:OBS].T], axis=1)
    eps_b = eps.transpose(1, 0, 2)

    out = pl.pallas_call(
        _knn_dec_kernel,
        grid=(B,),
        in_specs=[
            pl.BlockSpec((1, PRED, S_DIM), lambda b: (b, 0, 0)),
            pl.BlockSpec((1, S_DIM, OBS), lambda b: (b, 0, 0)),
            pl.BlockSpec((1, OBS, 2 * Z_DIM), lambda b: (b, 0, 0)),
            pl.BlockSpec((1, PRED, Z_DIM), lambda b: (b, 0, 0)),
            pl.BlockSpec((Z_DIM, H), lambda b: (0, 0)),
            pl.BlockSpec((1, H), lambda b: (0, 0)),
            pl.BlockSpec((H, XFLAT), lambda b: (0, 0)),
            pl.BlockSpec((1, XFLAT), lambda b: (0, 0)),
        ],
        out_specs=pl.BlockSpec((1, PRED, XFLAT), lambda b: (b, 0, 0)),
        out_shape=jax.ShapeDtypeStruct((B, PRED, XFLAT), f32),
    )(q, m_t, zt, eps_b, W_dec1, b_dec1.reshape(1, H), W_dec2,
      b_dec2.reshape(1, XFLAT))

    x_pred = out.transpose(1, 0, 2).reshape(PRED, B, 3, 8, 8)
    return x_pred


# speedup vs baseline: 23.6990x; 1.1109x over previous
import functools

import jax
import jax.numpy as jnp
from jax.experimental import pallas as pl
from jax.experimental.pallas import tpu as pltpu

B = 64
OBS = 2048
TOTAL = 2304
PRED = TOTAL - OBS
A_DIM = 5
S_DIM = 2
Z_DIM = 16
K = 5
DELTA = 1e-4
H = 512
XFLAT = 192

ENC_TILE = 1024


def _enc_kernel(x_ref, w1_ref, b1_ref, wms_ref, bms_ref, out_ref):
    x = x_ref[...] * 2.0 - 1.0
    h = jnp.tanh(jnp.dot(x, w1_ref[...], preferred_element_type=jnp.float32)
                 + b1_ref[...])
    y = jnp.dot(h, wms_ref[...], preferred_element_type=jnp.float32) + bms_ref[...]
    col = jax.lax.broadcasted_iota(jnp.int32, y.shape, 1)
    out_ref[...] = jnp.where(col < Z_DIM, y, jnp.exp(y))


def _scan_kernel(act_ref, wst_ref, w1_ref, b1_ref, w2_ref, b2_ref,
                 stx_ref, sty_ref, rx_ref, ry_ref):
    rx = jnp.zeros((TOTAL, B), jnp.float32)
    ry = jnp.zeros((TOTAL, B), jnp.float32)
    for a in range(A_DIM):
        act_a = act_ref[a]
        rx = rx + act_a * wst_ref[a, 0]
        ry = ry + act_a * wst_ref[a, 1]
    rx_ref[...] = rx
    ry_ref[...] = ry
    stx_ref[0:1, :] = jnp.zeros((1, B), jnp.float32)
    sty_ref[0:1, :] = jnp.zeros((1, B), jnp.float32)

    def step(t, st):
        stx, sty = st
        rpx = rx_ref[pl.ds(t, 1), :]
        rpy = ry_ref[pl.ds(t, 1), :]
        spx = stx + rpx
        spy = sty + rpy
        gx = jnp.zeros((1, B), jnp.float32) + b2_ref[0, 0]
        gy = jnp.zeros((1, B), jnp.float32) + b2_ref[0, 1]
        for k in range(10):
            hk = jnp.tanh(spx * w1_ref[0, k] + spy * w1_ref[1, k]
                          + b1_ref[0, k])
            gx = gx + hk * w2_ref[k, 0]
            gy = gy + hk * w2_ref[k, 1]
        stx = stx + rpx * jax.nn.sigmoid(gx)
        sty = sty + rpy * jax.nn.sigmoid(gy)
        stx_ref[pl.ds(t, 1), :] = stx
        sty_ref[pl.ds(t, 1), :] = sty
        return stx, sty

    jax.lax.fori_loop(1, TOTAL, step,
                      (jnp.zeros((1, B), jnp.float32),
                       jnp.zeros((1, B), jnp.float32)))


def _knn_kernel(q_ref, m_ref, zt_ref, ms_ref):
    q = q_ref[0]
    mt = m_ref[0]
    qx = q[:, 0:1]
    qy = q[:, 1:2]
    mx = mt[0:1, :]
    my = mt[1:2, :]
    dx = qx - mx
    dy = qy - my
    d = dx * dx + dy * dy

    u = jnp.zeros((PRED, OBS), jnp.float32)
    denom = jnp.zeros((PRED, 1), jnp.float32)
    for _ in range(K):
        mn = jnp.min(d, axis=1, keepdims=True)
        oneh = d <= mn
        wk = 1.0 / (mn + DELTA)
        u = u + jnp.where(oneh, wk, 0.0)
        denom = denom + wk
        d = jnp.where(oneh, jnp.float32(1e30), d)

    zt = zt_ref[0]
    numer = jnp.dot(u, zt, preferred_element_type=jnp.float32)
    ms_ref[0] = numer / denom


def _dec_kernel(ms_ref, eps_ref, wd1_ref, bd1_ref, wd2_ref, bd2_ref, out_ref):
    ms = jnp.transpose(ms_ref[...], (1, 0, 2))
    ms = ms.reshape(-1, 2 * Z_DIM)
    z = ms[:, :Z_DIM] + ms[:, Z_DIM:] * eps_ref[...]
    hd = jnp.tanh(jnp.dot(z, wd1_ref[...], preferred_element_type=jnp.float32)
                  + bd1_ref[...])
    xr = jnp.tanh(jnp.dot(hd, wd2_ref[...], preferred_element_type=jnp.float32)
                  + bd2_ref[...])
    out_ref[...] = (xr + 1.0) * 0.5


@jax.jit
def kernel(x_obs, actions, eps, W_enc1, b_enc1, W_mean, b_mean, W_std, b_std,
           W_st, W_sig1, b_sig1, W_sig2, b_sig2, W_dec1, b_dec1, W_dec2, b_dec2):
    f32 = jnp.float32
    nrows = B * OBS
    x_flat = x_obs.reshape(nrows, XFLAT)
    w_ms = jnp.concatenate([W_mean, W_std], axis=1)
    b_ms = jnp.concatenate([b_mean, b_std], axis=0).reshape(1, 2 * Z_DIM)
    b1r = b_enc1.reshape(1, H)

    zt = pl.pallas_call(
        _enc_kernel,
        grid=(nrows // ENC_TILE,),
        in_specs=[
            pl.BlockSpec((ENC_TILE, XFLAT), lambda i: (i, 0)),
            pl.BlockSpec((XFLAT, H), lambda i: (0, 0)),
            pl.BlockSpec((1, H), lambda i: (0, 0)),
            pl.BlockSpec((H, 2 * Z_DIM), lambda i: (0, 0)),
            pl.BlockSpec((1, 2 * Z_DIM), lambda i: (0, 0)),
        ],
        out_specs=pl.BlockSpec((ENC_TILE, 2 * Z_DIM), lambda i: (i, 0)),
        out_shape=jax.ShapeDtypeStruct((nrows, 2 * Z_DIM), f32),
    )(x_flat, W_enc1, b1r, w_ms, b_ms)
    zt = zt.reshape(B, OBS, 2 * Z_DIM)

    act_r = actions.transpose(1, 2, 0)
    smem_spec = pl.BlockSpec(memory_space=pltpu.SMEM)
    st_x, st_y = pl.pallas_call(
        _scan_kernel,
        grid=(1,),
        in_specs=[
            pl.BlockSpec((A_DIM, TOTAL, B), lambda i: (0, 0, 0)),
            smem_spec, smem_spec, smem_spec, smem_spec, smem_spec,
        ],
        out_specs=[pl.BlockSpec((TOTAL, B), lambda i: (0, 0)),
                   pl.BlockSpec((TOTAL, B), lambda i: (0, 0))],
        out_shape=[jax.ShapeDtypeStruct((TOTAL, B), f32),
                   jax.ShapeDtypeStruct((TOTAL, B), f32)],
        scratch_shapes=[pltpu.VMEM((TOTAL, B), f32),
                        pltpu.VMEM((TOTAL, B), f32)],
    )(act_r, W_st, W_sig1, b_sig1.reshape(1, 10), W_sig2,
      b_sig2.reshape(1, S_DIM))

    q = jnp.stack([st_x[OBS:], st_y[OBS:]], axis=-1).transpose(1, 0, 2)
    m_t = jnp.stack([st_x[:OBS].T, st_y[:OBS].T], axis=1)

    ms = pl.pallas_call(
        _knn_kernel,
        grid=(B,),
        in_specs=[
            pl.BlockSpec((1, PRED, S_DIM), lambda b: (b, 0, 0)),
            pl.BlockSpec((1, S_DIM, OBS), lambda b: (b, 0, 0)),
            pl.BlockSpec((1, OBS, 2 * Z_DIM), lambda b: (b, 0, 0)),
        ],
        out_specs=pl.BlockSpec((1, PRED, 2 * Z_DIM), lambda b: (b, 0, 0)),
        out_shape=jax.ShapeDtypeStruct((B, PRED, 2 * Z_DIM), f32),
    )(q, m_t, zt)

    PT = 64
    out = pl.pallas_call(
        _dec_kernel,
        grid=(PRED // PT,),
        in_specs=[
            pl.BlockSpec((B, PT, 2 * Z_DIM), lambda p: (0, p, 0)),
            pl.BlockSpec((PT * B, Z_DIM), lambda p: (p, 0)),
            pl.BlockSpec((Z_DIM, H), lambda p: (0, 0)),
            pl.BlockSpec((1, H), lambda p: (0, 0)),
            pl.BlockSpec((H, XFLAT), lambda p: (0, 0)),
            pl.BlockSpec((1, XFLAT), lambda p: (0, 0)),
        ],
        out_specs=pl.BlockSpec((PT * B, XFLAT), lambda p: (p, 0)),
        out_shape=jax.ShapeDtypeStruct((PRED * B, XFLAT), f32),
    )(ms, eps.reshape(PRED * B, Z_DIM), W_dec1, b_dec1.reshape(1, H),
      W_dec2, b_dec2.reshape(1, XFLAT))

    return out.reshape(PRED, B, 3, 8, 8)
